# baseline (speedup 1.0000x reference)
import functools

import jax
import jax.numpy as jnp
from jax import lax
from jax.experimental import pallas as pl
from jax.experimental.pallas import tpu as pltpu
from jax.experimental.pallas import tpu_sc as plsc

N = 10000
E = 320000
D = 128
H = 16
NC = 2
NS = 16
NW = NC * NS
EPW = E // NW
CH = 80
NCH = EPW // CH
EPS = E // NS
NCH2 = EPS // CH
NB = 5
NG = NCH // NB
NG2 = NCH2 // NB
NP = 10240
PT = NP // NS
SEG = NP // NW
NSEG = N // SEG
REM = N - (NW - 1) * SEG

_MESH = plsc.VectorSubcoreMesh(core_axis_name="c", subcore_axis_name="s")
_PARAMS = pltpu.CompilerParams(use_tc_tiling_on_sc=False)
_PARAMS_NL = pltpu.CompilerParams(use_tc_tiling_on_sc=False,
                                  needs_layout_passes=False)


@functools.partial(
    pl.kernel,
    out_type=jax.ShapeDtypeStruct((NP,), jnp.float32),
    mesh=_MESH,
    scratch_types=[
        pltpu.VMEM((EPS,), jnp.int32),
        pltpu.VMEM((NP,), jnp.float32),
        pltpu.VMEM((NS * PT,), jnp.float32),
        pltpu.VMEM((PT,), jnp.float32),
        pltpu.VMEM_SHARED((NS, NP), jnp.float32),
    ],
    compiler_params=_PARAMS_NL,
)
def _deg_kernel(ei3_hbm, zerosp_hbm, deg_hbm,
                didxf, accv, redv, aggsl, acc_sh):
    c = lax.axis_index("c")
    s = lax.axis_index("s")
    wid = s * NC + c
    pltpu.sync_copy(ei3_hbm.at[1, s], didxf)
    pltpu.sync_copy(zerosp_hbm, accv)
    ones16 = jnp.full((16,), 1.0, jnp.float32)

    def edge_chunk(k, _):
        base = pl.multiple_of(k * 80, 16)
        for u in range(5):
            di = didxf[pl.ds(base + u * 16, 16)]
            plsc.addupdate_scatter(accv, [di], ones16)
        return ()

    lax.fori_loop(0, EPS // 80, edge_chunk, ())

    pltpu.sync_copy(accv, acc_sh.at[s])
    plsc.subcore_barrier()
    for r in range(NS):
        pltpu.sync_copy(acc_sh.at[r, pl.ds(s * PT, PT)],
                        redv.at[pl.ds(r * PT, PT)])

    def red_chunk(j, _):
        off = pl.multiple_of(j * 16, 16)
        acc16 = redv[pl.ds(off, 16)]
        for r in range(1, NS):
            acc16 = acc16 + redv[pl.ds(r * PT + off, 16)]
        aggsl[pl.ds(off, 16)] = acc16
        return ()

    lax.fori_loop(0, PT // 16, red_chunk, ())
    pltpu.sync_copy(aggsl.at[pl.ds(c * (PT // 2), PT // 2)],
                    deg_hbm.at[pl.ds(s * PT + c * (PT // 2), PT // 2)])


@functools.partial(
    pl.kernel,
    out_type=jax.ShapeDtypeStruct((NC, NP, H), jnp.float32),
    mesh=_MESH,
    scratch_types=[
        pltpu.VMEM((NCH, CH), jnp.int32),
        pltpu.VMEM((NCH, CH), jnp.int32),
        pltpu.VMEM((NB, CH, H), jnp.float32),
        pltpu.SemaphoreType.DMA((NB,)),
        pltpu.SemaphoreType.DMA((NB,)),
        pltpu.VMEM_SHARED((NP, H), jnp.float32),
        pltpu.VMEM_SHARED((NP, H), jnp.float32),
    ],
    compiler_params=_PARAMS,
)
def _agg_rows_kernel(ei_hbm, g1_hbm, zeros2_hbm, aggp_hbm,
                     sidx, didx, rows, semg, sems, agg_sh, g1_sh):
    c = lax.axis_index("c")
    s = lax.axis_index("s")
    wid = s * NC + c
    pltpu.sync_copy(zeros2_hbm, agg_sh.at[pl.ds(s * PT, PT)])
    pltpu.sync_copy(ei_hbm.at[0, wid], sidx)
    pltpu.sync_copy(ei_hbm.at[1, wid], didx)
    @pl.when(s < NS - 1)
    def _():
        pltpu.sync_copy(g1_hbm.at[pl.ds(s * PT, PT)],
                        g1_sh.at[pl.ds(s * PT, PT)])

    @pl.when(s == NS - 1)
    def _():
        pltpu.sync_copy(g1_hbm.at[pl.ds((NS - 1) * PT, N - (NS - 1) * PT)],
                        g1_sh.at[pl.ds((NS - 1) * PT, N - (NS - 1) * PT)])
    plsc.subcore_barrier()

    def group(g, _):
        base = g * NB
        for b in range(NB):
            @pl.when(g > 0)
            def _():
                pltpu.make_async_copy(
                    rows.at[b], agg_sh.at[didx.at[base - NB + b]],
                    sems.at[b]).wait()
            pltpu.async_copy(g1_sh.at[sidx.at[base + b]], rows.at[b],
                             semg.at[b])
        for b in range(NB):
            pltpu.make_async_copy(g1_sh.at[sidx.at[base + b]], rows.at[b],
                                  semg.at[b]).wait()
            pltpu.async_copy(rows.at[b], agg_sh.at[didx.at[base + b]],
                             sems.at[b], add=True)
        return ()

    lax.fori_loop(0, NG, group, ())
    for b in range(NB):
        pltpu.make_async_copy(rows.at[b], agg_sh.at[didx.at[NCH - NB + b]],
                              sems.at[b]).wait()
    plsc.subcore_barrier()
    pltpu.sync_copy(agg_sh.at[pl.ds(s * PT, PT)],
                    aggp_hbm.at[c, pl.ds(s * PT, PT)])


@functools.partial(
    pl.kernel,
    out_type=jax.ShapeDtypeStruct((N,), jnp.float32),
    mesh=_MESH,
    scratch_types=[
        pltpu.VMEM((EPS,), jnp.int32),
        pltpu.VMEM((EPS,), jnp.int32),
        pltpu.VMEM((NP,), jnp.float32),
        pltpu.VMEM((NP,), jnp.float32),
        pltpu.VMEM((NS * PT,), jnp.float32),
        pltpu.VMEM((PT,), jnp.float32),
        pltpu.VMEM_SHARED((NS, NP), jnp.float32),
        pltpu.VMEM_SHARED((NP,), jnp.float32),
        pltpu.VMEM((SEG,), jnp.float32),
        pltpu.VMEM((SEG,), jnp.float32),
        pltpu.VMEM((SEG,), jnp.float32),
        pltpu.VMEM((SEG,), jnp.float32),
    ],
    compiler_params=_PARAMS_NL,
)
def _agg_scalar_kernel(ei3_hbm, g2_hbm, dinv_hbm, q_hbm, zerosp_hbm,
                       out_hbm,
                       sidxf, didxf, g2v, accv, redv, aggsl, acc_sh, agg_sh,
                       aggv, dinvv, qv, outv):
    c = lax.axis_index("c")
    s = lax.axis_index("s")
    wid = s * NC + c
    pltpu.sync_copy(g2_hbm, g2v.at[pl.ds(0, N)])
    pltpu.sync_copy(ei3_hbm.at[0, s], sidxf)
    pltpu.sync_copy(ei3_hbm.at[1, s], didxf)
    pltpu.sync_copy(zerosp_hbm, accv)

    def edge_chunk(k, _):
        base = pl.multiple_of(k * 80, 16)
        for u in range(5):
            off = base + u * 16
            si = sidxf[pl.ds(off, 16)]
            di = didxf[pl.ds(off, 16)]
            v = plsc.load_gather(g2v, [si])
            plsc.addupdate_scatter(accv, [di], v)
        return ()

    lax.fori_loop(0, EPS // 80, edge_chunk, ())

    pltpu.sync_copy(accv, acc_sh.at[s])
    plsc.subcore_barrier()
    for r in range(NS):
        pltpu.sync_copy(acc_sh.at[r, pl.ds(s * PT, PT)],
                        redv.at[pl.ds(r * PT, PT)])

    def red_chunk(j, _):
        off = pl.multiple_of(j * 16, 16)
        acc16 = redv[pl.ds(off, 16)]
        for r in range(1, NS):
            acc16 = acc16 + redv[pl.ds(r * PT + off, 16)]
        aggsl[pl.ds(off, 16)] = acc16
        return ()

    lax.fori_loop(0, PT // 16, red_chunk, ())
    pltpu.sync_copy(aggsl, agg_sh.at[pl.ds(s * PT, PT)])
    plsc.subcore_barrier()

    pltpu.sync_copy(agg_sh.at[pl.ds(wid * SEG, SEG)], aggv)

    def combine(n_chunks):
        def body(k, _):
            off = pl.multiple_of(k * 16, 16)
            a = aggv[pl.ds(off, 16)]
            dv = dinvv[pl.ds(off, 16)]
            q = qv[pl.ds(off, 16)]
            outv[pl.ds(off, 16)] = a * dv + q
            return ()
        lax.fori_loop(0, n_chunks, body, ())

    @pl.when(wid < NW - 1)
    def _():
        pltpu.sync_copy(dinv_hbm.at[pl.ds(wid * SEG, SEG)], dinvv)
        pltpu.sync_copy(q_hbm.at[pl.ds(wid * SEG, SEG)], qv)
        combine(SEG // 16)
        pltpu.sync_copy(outv, out_hbm.at[pl.ds(wid * SEG, SEG)])

    @pl.when(wid == NW - 1)
    def _():
        pltpu.sync_copy(dinv_hbm.at[pl.ds((NW - 1) * SEG, REM)],
                        dinvv.at[pl.ds(0, REM)])
        pltpu.sync_copy(q_hbm.at[pl.ds((NW - 1) * SEG, REM)],
                        qv.at[pl.ds(0, REM)])
        combine(REM // 16)
        pltpu.sync_copy(outv.at[pl.ds(0, REM)],
                        out_hbm.at[pl.ds((NW - 1) * SEG, REM)])


def _tc1_body(x_ref, w1_ref, deg_ref, g1_ref, dinv_ref):
    dinv = lax.rsqrt(deg_ref[...][:N] + 1.0)
    h = jnp.dot(x_ref[...], w1_ref[...], preferred_element_type=jnp.float32)
    g1_ref[...] = h * dinv
    dinv_ref[...] = dinv


def _tc2_body(aggp_ref, g1_ref, dinv_ref, b1_ref, w2_ref, b2_ref,
              g2_ref, q_ref):
    dinv = dinv_ref[...]
    agg = aggp_ref[0, :N] + aggp_ref[1, :N] + g1_ref[...]
    r1 = jnp.maximum(agg * dinv + b1_ref[...], 0.0)
    g2 = jnp.dot(r1, w2_ref[...], preferred_element_type=jnp.float32) * dinv
    g2_ref[...] = g2
    q_ref[...] = g2 * dinv + b2_ref[...]


def kernel(x, edge_index, W1, b1, W2, b2):
    ei32 = edge_index.astype(jnp.int32)
    ei = ei32.reshape(2, NW, NCH, CH)
    ei2 = ei32.reshape(2, NS, NCH2, CH)
    ei3 = ei32.reshape(2, NS, EPS)

    zeros2 = jnp.zeros((PT, H), jnp.float32)
    zerosp = jnp.zeros((NP,), jnp.float32)

    deg = _deg_kernel(ei3, zerosp)

    g1, dinv = pl.pallas_call(
        _tc1_body,
        out_shape=[
            jax.ShapeDtypeStruct((N, H), jnp.float32),
            jax.ShapeDtypeStruct((N, 1), jnp.float32),
        ],
    )(x, W1, deg.reshape(NP, 1))

    aggp = _agg_rows_kernel(ei, g1, zeros2)

    g2, q = pl.pallas_call(
        _tc2_body,
        out_shape=[
            jax.ShapeDtypeStruct((N, 1), jnp.float32),
            jax.ShapeDtypeStruct((N, 1), jnp.float32),
        ],
    )(aggp, g1, dinv, b1.reshape(1, H), W2, b2.reshape(1, 1))

    out = _agg_scalar_kernel(ei3, g2.reshape(N), dinv.reshape(N),
                             q.reshape(N), zerosp)
    return out.reshape(N, 1)


# speedup vs baseline: 1.0330x; 1.0330x over previous
import functools

import jax
import jax.numpy as jnp
from jax import lax
from jax.experimental import pallas as pl
from jax.experimental.pallas import tpu as pltpu
from jax.experimental.pallas import tpu_sc as plsc

N = 10000
E = 320000
D = 128
H = 16
NC = 2
NS = 16
NW = NC * NS
EPW = E // NW
CH = 80
NCH = EPW // CH
EPS = E // NS
NCH2 = EPS // CH
NB = 5
NG = NCH // NB
NG2 = NCH2 // NB
NP = 10240
PT = NP // NS
SEG = NP // NW
NSEG = N // SEG
REM = N - (NW - 1) * SEG

_MESH = plsc.VectorSubcoreMesh(core_axis_name="c", subcore_axis_name="s")
_PARAMS = pltpu.CompilerParams(use_tc_tiling_on_sc=False)
_PARAMS_NL = pltpu.CompilerParams(use_tc_tiling_on_sc=False,
                                  needs_layout_passes=False)


@functools.partial(
    pl.kernel,
    out_type=jax.ShapeDtypeStruct((NP,), jnp.float32),
    mesh=_MESH,
    scratch_types=[
        pltpu.VMEM((NCH2, CH), jnp.int32),
        pltpu.VMEM((CH,), jnp.float32),
        pltpu.SemaphoreType.DMA,
        pltpu.VMEM_SHARED((NP,), jnp.float32),
    ],
    compiler_params=_PARAMS,
)
def _deg_kernel(ei2_hbm, zeros1_hbm, ones_hbm, deg_hbm,
                didx, ones_v, sem, deg_sh):
    c = lax.axis_index("c")
    s = lax.axis_index("s")
    wid = s * NC + c
    pltpu.sync_copy(zeros1_hbm, deg_sh.at[pl.ds(s * PT, PT)])
    pltpu.sync_copy(ones_hbm, ones_v)
    pltpu.sync_copy(ei2_hbm.at[1, s], didx)
    plsc.subcore_barrier()

    def fire(i, _):
        pltpu.async_copy(ones_v, deg_sh.at[didx.at[i]], sem, add=True)
        return ()

    def drain(i, _):
        pltpu.make_async_copy(ones_v, deg_sh.at[didx.at[i]], sem).wait()
        return ()

    lax.fori_loop(0, NCH2, fire, ())
    lax.fori_loop(0, NCH2, drain, ())
    plsc.subcore_barrier()
    pltpu.sync_copy(deg_sh.at[pl.ds(wid * SEG, SEG)],
                    deg_hbm.at[pl.ds(wid * SEG, SEG)])


@functools.partial(
    pl.kernel,
    out_type=jax.ShapeDtypeStruct((NC, NP, H), jnp.float32),
    mesh=_MESH,
    scratch_types=[
        pltpu.VMEM((NCH, CH), jnp.int32),
        pltpu.VMEM((NCH, CH), jnp.int32),
        pltpu.VMEM((NB, CH, H), jnp.float32),
        pltpu.SemaphoreType.DMA((NB,)),
        pltpu.SemaphoreType.DMA((NB,)),
        pltpu.VMEM_SHARED((NP, H), jnp.float32),
        pltpu.VMEM_SHARED((NP, H), jnp.float32),
    ],
    compiler_params=_PARAMS,
)
def _agg_rows_kernel(ei_hbm, g1_hbm, zeros2_hbm, aggp_hbm,
                     sidx, didx, rows, semg, sems, agg_sh, g1_sh):
    c = lax.axis_index("c")
    s = lax.axis_index("s")
    wid = s * NC + c
    pltpu.sync_copy(zeros2_hbm, agg_sh.at[pl.ds(s * PT, PT)])
    pltpu.sync_copy(ei_hbm.at[0, wid], sidx)
    pltpu.sync_copy(ei_hbm.at[1, wid], didx)
    @pl.when(s < NS - 1)
    def _():
        pltpu.sync_copy(g1_hbm.at[pl.ds(s * PT, PT)],
                        g1_sh.at[pl.ds(s * PT, PT)])

    @pl.when(s == NS - 1)
    def _():
        pltpu.sync_copy(g1_hbm.at[pl.ds((NS - 1) * PT, N - (NS - 1) * PT)],
                        g1_sh.at[pl.ds((NS - 1) * PT, N - (NS - 1) * PT)])
    plsc.subcore_barrier()

    def group(g, _):
        base = g * NB
        for b in range(NB):
            @pl.when(g > 0)
            def _():
                pltpu.make_async_copy(
                    rows.at[b], agg_sh.at[didx.at[base - NB + b]],
                    sems.at[b]).wait()
            pltpu.async_copy(g1_sh.at[sidx.at[base + b]], rows.at[b],
                             semg.at[b])
        for b in range(NB):
            pltpu.make_async_copy(g1_sh.at[sidx.at[base + b]], rows.at[b],
                                  semg.at[b]).wait()
            pltpu.async_copy(rows.at[b], agg_sh.at[didx.at[base + b]],
                             sems.at[b], add=True)
        return ()

    lax.fori_loop(0, NG, group, ())
    for b in range(NB):
        pltpu.make_async_copy(rows.at[b], agg_sh.at[didx.at[NCH - NB + b]],
                              sems.at[b]).wait()
    plsc.subcore_barrier()
    pltpu.sync_copy(agg_sh.at[pl.ds(s * PT, PT)],
                    aggp_hbm.at[c, pl.ds(s * PT, PT)])


@functools.partial(
    pl.kernel,
    out_type=jax.ShapeDtypeStruct((N,), jnp.float32),
    mesh=_MESH,
    scratch_types=[
        pltpu.VMEM((NCH2, CH), jnp.int32),
        pltpu.VMEM((NCH2, CH), jnp.int32),
        pltpu.VMEM((NB, CH), jnp.float32),
        pltpu.SemaphoreType.DMA((NB,)),
        pltpu.SemaphoreType.DMA((NB,)),
        pltpu.VMEM_SHARED((NP,), jnp.float32),
        pltpu.VMEM_SHARED((NP,), jnp.float32),
        pltpu.VMEM((SEG,), jnp.float32),
        pltpu.VMEM((SEG,), jnp.float32),
        pltpu.VMEM((SEG,), jnp.float32),
        pltpu.VMEM((SEG,), jnp.float32),
    ],
    compiler_params=_PARAMS,
)
def _agg_scalar_kernel(ei2_hbm, g2_hbm, dinv_hbm, q_hbm, zeros1_hbm,
                       out_hbm,
                       sidx, didx, vals, semg, sems, agg_sh, g2_sh,
                       aggv, dinvv, qv, outv):
    c = lax.axis_index("c")
    s = lax.axis_index("s")
    wid = s * NC + c
    pltpu.sync_copy(zeros1_hbm, agg_sh.at[pl.ds(s * PT, PT)])
    pltpu.sync_copy(ei2_hbm.at[0, s], sidx)
    pltpu.sync_copy(ei2_hbm.at[1, s], didx)

    @pl.when(s < NS - 1)
    def _():
        pltpu.sync_copy(g2_hbm.at[pl.ds(s * PT, PT)],
                        g2_sh.at[pl.ds(s * PT, PT)])

    @pl.when(s == NS - 1)
    def _():
        pltpu.sync_copy(g2_hbm.at[pl.ds((NS - 1) * PT, N - (NS - 1) * PT)],
                        g2_sh.at[pl.ds((NS - 1) * PT, N - (NS - 1) * PT)])
    plsc.subcore_barrier()

    def group(g, _):
        base = g * NB
        for b in range(NB):
            @pl.when(g > 0)
            def _():
                pltpu.make_async_copy(
                    vals.at[b], agg_sh.at[didx.at[base - NB + b]],
                    sems.at[b]).wait()
            pltpu.async_copy(g2_sh.at[sidx.at[base + b]], vals.at[b],
                             semg.at[b])
        for b in range(NB):
            pltpu.make_async_copy(g2_sh.at[sidx.at[base + b]], vals.at[b],
                                  semg.at[b]).wait()
            pltpu.async_copy(vals.at[b], agg_sh.at[didx.at[base + b]],
                             sems.at[b], add=True)
        return ()

    lax.fori_loop(0, NG2, group, ())
    for b in range(NB):
        pltpu.make_async_copy(vals.at[b], agg_sh.at[didx.at[NCH2 - NB + b]],
                              sems.at[b]).wait()
    plsc.subcore_barrier()

    pltpu.sync_copy(agg_sh.at[pl.ds(wid * SEG, SEG)], aggv)

    def combine(n_chunks):
        def body(k, _):
            off = pl.multiple_of(k * 16, 16)
            a = aggv[pl.ds(off, 16)]
            dv = dinvv[pl.ds(off, 16)]
            q = qv[pl.ds(off, 16)]
            outv[pl.ds(off, 16)] = a * dv + q
            return ()
        lax.fori_loop(0, n_chunks, body, ())

    @pl.when(wid < NW - 1)
    def _():
        pltpu.sync_copy(dinv_hbm.at[pl.ds(wid * SEG, SEG)], dinvv)
        pltpu.sync_copy(q_hbm.at[pl.ds(wid * SEG, SEG)], qv)
        combine(SEG // 16)
        pltpu.sync_copy(outv, out_hbm.at[pl.ds(wid * SEG, SEG)])

    @pl.when(wid == NW - 1)
    def _():
        pltpu.sync_copy(dinv_hbm.at[pl.ds((NW - 1) * SEG, REM)],
                        dinvv.at[pl.ds(0, REM)])
        pltpu.sync_copy(q_hbm.at[pl.ds((NW - 1) * SEG, REM)],
                        qv.at[pl.ds(0, REM)])
        combine(REM // 16)
        pltpu.sync_copy(outv.at[pl.ds(0, REM)],
                        out_hbm.at[pl.ds((NW - 1) * SEG, REM)])


def _tc1_body(x_ref, w1_ref, deg_ref, g1_ref, dinv_ref):
    dinv = lax.rsqrt(deg_ref[...][:N] + 1.0)
    h = jnp.dot(x_ref[...], w1_ref[...], preferred_element_type=jnp.float32)
    g1_ref[...] = h * dinv
    dinv_ref[...] = dinv


def _tc2_body(aggp_ref, g1_ref, dinv_ref, b1_ref, w2_ref, b2_ref,
              g2_ref, q_ref):
    dinv = dinv_ref[...]
    agg = aggp_ref[0, :N] + aggp_ref[1, :N] + g1_ref[...]
    r1 = jnp.maximum(agg * dinv + b1_ref[...], 0.0)
    g2 = jnp.dot(r1, w2_ref[...], preferred_element_type=jnp.float32) * dinv
    g2_ref[...] = g2
    q_ref[...] = g2 * dinv + b2_ref[...]


def kernel(x, edge_index, W1, b1, W2, b2):
    ei32 = edge_index.astype(jnp.int32)
    ei = ei32.reshape(2, NW, NCH, CH)
    ei2 = ei32.reshape(2, NS, NCH2, CH)
    ei3 = ei32.reshape(2, NS, EPS)

    zeros2 = jnp.zeros((PT, H), jnp.float32)
    zeros1 = jnp.zeros((PT,), jnp.float32)
    ones = jnp.ones((CH,), jnp.float32)

    deg = _deg_kernel(ei2, zeros1, ones)

    g1, dinv = pl.pallas_call(
        _tc1_body,
        out_shape=[
            jax.ShapeDtypeStruct((N, H), jnp.float32),
            jax.ShapeDtypeStruct((N, 1), jnp.float32),
        ],
    )(x, W1, deg.reshape(NP, 1))

    aggp = _agg_rows_kernel(ei, g1, zeros2)

    g2, q = pl.pallas_call(
        _tc2_body,
        out_shape=[
            jax.ShapeDtypeStruct((N, 1), jnp.float32),
            jax.ShapeDtypeStruct((N, 1), jnp.float32),
        ],
    )(aggp, g1, dinv, b1.reshape(1, H), W2, b2.reshape(1, 1))

    out = _agg_scalar_kernel(ei2, g2.reshape(N), dinv.reshape(N),
                             q.reshape(N), zeros1)
    return out.reshape(N, 1)


# speedup vs baseline: 1.0606x; 1.0268x over previous
import functools

import jax
import jax.numpy as jnp
from jax import lax
from jax.experimental import pallas as pl
from jax.experimental.pallas import tpu as pltpu
from jax.experimental.pallas import tpu_sc as plsc

N = 10000
E = 320000
D = 128
H = 16
NC = 2
NS = 16
NW = NC * NS
EPW = E // NW
CH = 80
NCH = EPW // CH
EPS = E // NS
NCH2 = EPS // CH
NB = 5
NG = NCH // NB
NG2 = NCH2 // NB
NP = 10240
PT = NP // NS
SEG = NP // NW
NSEG = N // SEG
REM = N - (NW - 1) * SEG

_MESH = plsc.VectorSubcoreMesh(core_axis_name="c", subcore_axis_name="s")
_PARAMS = pltpu.CompilerParams(use_tc_tiling_on_sc=False)
_PARAMS_NL = pltpu.CompilerParams(use_tc_tiling_on_sc=False,
                                  needs_layout_passes=False)


@functools.partial(
    pl.kernel,
    out_type=jax.ShapeDtypeStruct((NP,), jnp.float32),
    mesh=_MESH,
    scratch_types=[
        pltpu.VMEM((NCH2, CH), jnp.int32),
        pltpu.VMEM((CH,), jnp.float32),
        pltpu.SemaphoreType.DMA,
        pltpu.VMEM_SHARED((NP,), jnp.float32),
    ],
    compiler_params=_PARAMS,
)
def _deg_kernel(ei2_hbm, zeros1_hbm, ones_hbm, deg_hbm,
                didx, ones_v, sem, deg_sh):
    c = lax.axis_index("c")
    s = lax.axis_index("s")
    wid = s * NC + c
    pltpu.sync_copy(zeros1_hbm, deg_sh.at[pl.ds(s * PT, PT)])
    pltpu.sync_copy(ones_hbm, ones_v)
    pltpu.sync_copy(ei2_hbm.at[1, s], didx)
    plsc.subcore_barrier()

    def fire(i, _):
        pltpu.async_copy(ones_v, deg_sh.at[didx.at[i]], sem, add=True)
        return ()

    def drain(i, _):
        pltpu.make_async_copy(ones_v, deg_sh.at[didx.at[i]], sem).wait()
        return ()

    lax.fori_loop(0, NCH2, fire, ())
    lax.fori_loop(0, NCH2, drain, ())
    plsc.subcore_barrier()
    pltpu.sync_copy(deg_sh.at[pl.ds(wid * SEG, SEG)],
                    deg_hbm.at[pl.ds(wid * SEG, SEG)])


@functools.partial(
    pl.kernel,
    out_type=jax.ShapeDtypeStruct((NC, NP, H), jnp.float32),
    mesh=_MESH,
    scratch_types=[
        pltpu.VMEM((NCH, CH), jnp.int32),
        pltpu.VMEM((NCH, CH), jnp.int32),
        pltpu.VMEM((NB, CH, H), jnp.float32),
        pltpu.SemaphoreType.DMA((NB,)),
        pltpu.SemaphoreType.DMA((NB,)),
        pltpu.VMEM_SHARED((NP, H), jnp.float32),
        pltpu.VMEM_SHARED((NP, H), jnp.float32),
    ],
    compiler_params=_PARAMS,
)
def _agg_rows_kernel(ei_hbm, g1_hbm, zeros2_hbm, aggp_hbm,
                     sidx, didx, rows, semg, sems, agg_sh, g1_sh):
    c = lax.axis_index("c")
    s = lax.axis_index("s")
    wid = s * NC + c
    pltpu.sync_copy(zeros2_hbm, agg_sh.at[pl.ds(s * PT, PT)])
    pltpu.sync_copy(ei_hbm.at[0, wid], sidx)
    pltpu.sync_copy(ei_hbm.at[1, wid], didx)
    @pl.when(s < NS - 1)
    def _():
        pltpu.sync_copy(g1_hbm.at[pl.ds(s * PT, PT)],
                        g1_sh.at[pl.ds(s * PT, PT)])

    @pl.when(s == NS - 1)
    def _():
        pltpu.sync_copy(g1_hbm.at[pl.ds((NS - 1) * PT, N - (NS - 1) * PT)],
                        g1_sh.at[pl.ds((NS - 1) * PT, N - (NS - 1) * PT)])
    plsc.subcore_barrier()

    def group(g, _):
        base = g * NB
        for b in range(NB):
            @pl.when(g > 0)
            def _():
                pltpu.make_async_copy(
                    rows.at[b], agg_sh.at[didx.at[base - NB + b]],
                    sems.at[b]).wait()
            pltpu.async_copy(g1_sh.at[sidx.at[base + b]], rows.at[b],
                             semg.at[b])
        for b in range(NB):
            pltpu.make_async_copy(g1_sh.at[sidx.at[base + b]], rows.at[b],
                                  semg.at[b]).wait()
            pltpu.async_copy(rows.at[b], agg_sh.at[didx.at[base + b]],
                             sems.at[b], add=True)
        return ()

    lax.fori_loop(0, NG, group, ())
    for b in range(NB):
        pltpu.make_async_copy(rows.at[b], agg_sh.at[didx.at[NCH - NB + b]],
                              sems.at[b]).wait()
    plsc.subcore_barrier()
    pltpu.sync_copy(agg_sh.at[pl.ds(s * PT, PT)],
                    aggp_hbm.at[c, pl.ds(s * PT, PT)])


@functools.partial(
    pl.kernel,
    out_type=jax.ShapeDtypeStruct((N,), jnp.float32),
    mesh=_MESH,
    scratch_types=[
        pltpu.VMEM((NCH2, CH), jnp.int32),
        pltpu.VMEM((NCH2, CH), jnp.int32),
        pltpu.VMEM((NB, CH), jnp.float32),
        pltpu.SemaphoreType.DMA((NB,)),
        pltpu.SemaphoreType.DMA((NB,)),
        pltpu.VMEM_SHARED((NP,), jnp.float32),
        pltpu.VMEM_SHARED((NP,), jnp.float32),
        pltpu.VMEM((SEG,), jnp.float32),
        pltpu.VMEM((SEG,), jnp.float32),
        pltpu.VMEM((SEG,), jnp.float32),
        pltpu.VMEM((SEG,), jnp.float32),
        pltpu.VMEM((16,), jnp.float32),
    ],
    compiler_params=_PARAMS,
)
def _agg_scalar_kernel(ei2_hbm, g2_hbm, dinv_hbm, b2v_hbm, zeros1_hbm,
                       out_hbm,
                       sidx, didx, vals, semg, sems, agg_sh, g2_sh,
                       aggv, dinvv, g2v, outv, b2l):
    c = lax.axis_index("c")
    s = lax.axis_index("s")
    wid = s * NC + c
    pltpu.sync_copy(zeros1_hbm, agg_sh.at[pl.ds(s * PT, PT)])
    pltpu.sync_copy(ei2_hbm.at[0, s], sidx)
    pltpu.sync_copy(ei2_hbm.at[1, s], didx)

    @pl.when(s < NS - 1)
    def _():
        pltpu.sync_copy(g2_hbm.at[pl.ds(s * PT, PT)],
                        g2_sh.at[pl.ds(s * PT, PT)])

    @pl.when(s == NS - 1)
    def _():
        pltpu.sync_copy(g2_hbm.at[pl.ds((NS - 1) * PT, N - (NS - 1) * PT)],
                        g2_sh.at[pl.ds((NS - 1) * PT, N - (NS - 1) * PT)])
    plsc.subcore_barrier()

    def group(g, _):
        base = g * NB
        for b in range(NB):
            @pl.when(g > 0)
            def _():
                pltpu.make_async_copy(
                    vals.at[b], agg_sh.at[didx.at[base - NB + b]],
                    sems.at[b]).wait()
            pltpu.async_copy(g2_sh.at[sidx.at[base + b]], vals.at[b],
                             semg.at[b])
        for b in range(NB):
            pltpu.make_async_copy(g2_sh.at[sidx.at[base + b]], vals.at[b],
                                  semg.at[b]).wait()
            pltpu.async_copy(vals.at[b], agg_sh.at[didx.at[base + b]],
                             sems.at[b], add=True)
        return ()

    lax.fori_loop(0, NG2, group, ())
    for b in range(NB):
        pltpu.make_async_copy(vals.at[b], agg_sh.at[didx.at[NCH2 - NB + b]],
                              sems.at[b]).wait()
    plsc.subcore_barrier()

    pltpu.sync_copy(agg_sh.at[pl.ds(wid * SEG, SEG)], aggv)
    pltpu.sync_copy(b2v_hbm, b2l)

    def combine(n_chunks):
        def body(k, _):
            off = pl.multiple_of(k * 16, 16)
            a = aggv[pl.ds(off, 16)]
            dv = dinvv[pl.ds(off, 16)]
            g = g2v[pl.ds(off, 16)]
            outv[pl.ds(off, 16)] = (a + g) * dv + b2l[...]
            return ()
        lax.fori_loop(0, n_chunks, body, ())

    @pl.when(wid < NW - 1)
    def _():
        pltpu.sync_copy(dinv_hbm.at[pl.ds(wid * SEG, SEG)], dinvv)
        pltpu.sync_copy(g2_sh.at[pl.ds(wid * SEG, SEG)], g2v)
        combine(SEG // 16)
        pltpu.sync_copy(outv, out_hbm.at[pl.ds(wid * SEG, SEG)])

    @pl.when(wid == NW - 1)
    def _():
        pltpu.sync_copy(dinv_hbm.at[pl.ds((NW - 1) * SEG, REM)],
                        dinvv.at[pl.ds(0, REM)])
        pltpu.sync_copy(g2_sh.at[pl.ds((NW - 1) * SEG, REM)],
                        g2v.at[pl.ds(0, REM)])
        combine(REM // 16)
        pltpu.sync_copy(outv.at[pl.ds(0, REM)],
                        out_hbm.at[pl.ds((NW - 1) * SEG, REM)])


def _tc1a_body(x_ref, w1_ref, h_ref):
    h_ref[...] = jnp.dot(x_ref[...], w1_ref[...],
                         preferred_element_type=jnp.float32)


def _tc1b_body(h_ref, deg_ref, g1_ref, dinv_ref):
    dinv = lax.rsqrt(deg_ref[...][:N] + 1.0)
    g1_ref[...] = h_ref[...] * dinv
    dinv_ref[...] = dinv


def _tc2_body(aggp_ref, g1_ref, dinv_ref, b1_ref, w2_ref, g2_ref):
    dinv = dinv_ref[...]
    agg = aggp_ref[0, :N] + aggp_ref[1, :N] + g1_ref[...]
    r1 = jnp.maximum(agg * dinv + b1_ref[...], 0.0)
    g2_ref[...] = jnp.dot(r1, w2_ref[...],
                          preferred_element_type=jnp.float32) * dinv


def kernel(x, edge_index, W1, b1, W2, b2):
    ei32 = edge_index.astype(jnp.int32)
    ei = ei32.reshape(2, NW, NCH, CH)
    ei2 = ei32.reshape(2, NS, NCH2, CH)
    ei3 = ei32.reshape(2, NS, EPS)

    zeros2 = jnp.zeros((PT, H), jnp.float32)
    zeros1 = jnp.zeros((PT,), jnp.float32)
    ones = jnp.ones((CH,), jnp.float32)

    b2v = jnp.broadcast_to(b2.reshape(1), (16,))

    h = pl.pallas_call(
        _tc1a_body,
        out_shape=jax.ShapeDtypeStruct((N, H), jnp.float32),
    )(x, W1)

    deg = _deg_kernel(ei2, zeros1, ones)

    g1, dinv = pl.pallas_call(
        _tc1b_body,
        out_shape=[
            jax.ShapeDtypeStruct((N, H), jnp.float32),
            jax.ShapeDtypeStruct((N, 1), jnp.float32),
        ],
    )(h, deg.reshape(NP, 1))

    aggp = _agg_rows_kernel(ei, g1, zeros2)

    g2 = pl.pallas_call(
        _tc2_body,
        out_shape=jax.ShapeDtypeStruct((N, 1), jnp.float32),
    )(aggp, g1, dinv, b1.reshape(1, H), W2)

    out = _agg_scalar_kernel(ei2, g2.reshape(N), dinv.reshape(N),
                             b2v, zeros1)
    return out.reshape(N, 1)
